# Initial kernel scaffold; baseline (speedup 1.0000x reference)
import jax
import jax.numpy as jnp
from jax.experimental import pallas as pl


def kernel(x, edge_index, W_in, b_in, ln1_g, ln1_b, pos, Wm1, bm1, Wm2, bm2, Wih, bih, Whh, bhh, ln2_g, ln2_b, Wo, bo):
    raise NotImplementedError("write your pallas kernel here")


# speedup vs baseline: 3.2295x; 3.2295x over previous
import functools

import jax
import jax.numpy as jnp
from jax import lax
from jax.experimental import pallas as pl
from jax.experimental.pallas import tpu as pltpu
from jax.experimental.pallas import tpu_sc as plsc

N = 10368
E = 207360
H = 192
HC = 128
NSTEPS = 4
NCORES = 2
NSUB = 16
EPT = E // NSUB
K = 96
NCHUNK = EPT // K
G = 45
NGRP = NCHUNK // G
RPT = N // NSUB
NBLK = 8
BLK = N // NBLK
LN_EPS = 1e-5

_sc_mesh = plsc.VectorSubcoreMesh(core_axis_name="c", subcore_axis_name="s")


@functools.partial(
    pl.kernel,
    out_type=jax.ShapeDtypeStruct((NCORES, N, HC), jnp.float32),
    mesh=_sc_mesh,
    scratch_types=[
        pltpu.VMEM((G, K), jnp.int32),
        pltpu.VMEM((G, K), jnp.int32),
        pltpu.VMEM((K, HC), jnp.float32),
        pltpu.VMEM((K, HC), jnp.float32),
        pltpu.VMEM_SHARED((N, HC), jnp.float32),
        pltpu.SemaphoreType.DMA,
        pltpu.SemaphoreType.DMA,
    ],
)
def _edge_pass(ha0, ha1, hb0, hb1, srcr, dstr, out, si, di, ar, br, acc,
               s1, s2):
    cid = lax.axis_index("c")
    sid = lax.axis_index("s")
    row0 = sid * RPT

    zero16 = jnp.zeros((16,), jnp.float32)

    def zrow(i, carry):
        r = i // (HC // 16)
        c = (i % (HC // 16)) * 16
        ar[r, pl.ds(c, 16)] = zero16
        return carry

    lax.fori_loop(0, K * (HC // 16), zrow, 0)
    for j in range(RPT // K):
        pltpu.sync_copy(ar, acc.at[pl.ds(row0 + j * K, K)])
    rem = RPT - (RPT // K) * K
    if rem:
        pltpu.sync_copy(ar.at[pl.ds(0, rem)],
                        acc.at[pl.ds(row0 + (RPT // K) * K, rem)])
    plsc.subcore_barrier()

    def chunk(c, carry):
        @pl.when(cid == 0)
        def _():
            d1 = pltpu.async_copy(ha0.at[si.at[c]], ar, s1)
            d2 = pltpu.async_copy(hb0.at[di.at[c]], br, s2)
            d1.wait()
            d2.wait()

        @pl.when(cid == 1)
        def _():
            d1 = pltpu.async_copy(ha1.at[si.at[c]], ar, s1)
            d2 = pltpu.async_copy(hb1.at[di.at[c]], br, s2)
            d1.wait()
            d2.wait()

        def rowfn(r, carry2):
            for cc in range(HC // 16):
                a = ar[r, pl.ds(cc * 16, 16)]
                b = br[r, pl.ds(cc * 16, 16)]
                v = jnp.maximum(a + b, 0.0)
                u = lax.bitcast_convert_type(v, jnp.int32)
                u = u & jnp.int32(-65536)
                ar[r, pl.ds(cc * 16, 16)] = lax.bitcast_convert_type(
                    u, jnp.float32)
            return carry2

        lax.fori_loop(0, K, rowfn, 0)
        pltpu.sync_copy(ar, acc.at[di.at[c]], add=True)
        return carry

    for g in range(NGRP):
        pltpu.sync_copy(srcr.at[sid, g], si)
        pltpu.sync_copy(dstr.at[sid, g], di)
        lax.fori_loop(0, G, chunk, 0)
    plsc.subcore_barrier()
    pltpu.sync_copy(acc.at[pl.ds(row0, RPT)], out.at[cid, pl.ds(row0, RPT)])


def _whole(shape):
    nd = len(shape)
    return pl.BlockSpec(shape, lambda i, _n=nd: (0,) * _n)


def _blocked():
    return pl.BlockSpec((BLK, H), lambda i: (i, 0))


def _split_in():
    return pl.BlockSpec((NCORES, BLK, HC), lambda i: (0, i, 0))


def _hc_blocked():
    return pl.BlockSpec((BLK, HC), lambda i: (i, 0))


def _dotbf(a, b):
    return jnp.dot(a.astype(jnp.bfloat16), b.astype(jnp.bfloat16),
                   preferred_element_type=jnp.float32)


def _dot_exact(a, b):
    a = a.astype(jnp.float32)
    hi = a.astype(jnp.bfloat16)
    mid = (a - hi.astype(jnp.float32)).astype(jnp.bfloat16)
    lo = (a - hi.astype(jnp.float32)
          - mid.astype(jnp.float32)).astype(jnp.bfloat16)
    bb = b.astype(jnp.bfloat16)
    f32 = jnp.float32
    return (jnp.dot(hi, bb, preferred_element_type=f32)
            + jnp.dot(mid, bb, preferred_element_type=f32)
            + jnp.dot(lo, bb, preferred_element_type=f32))


def _matin_body(x_ref, win, bin_, t_o):
    t_o[...] = _dotbf(x_ref[...], win[...]) + bin_[...]


_matin = pl.pallas_call(
    _matin_body,
    grid=(NBLK,),
    in_specs=[
        pl.BlockSpec((BLK, 10), lambda i: (i, 0)),
        _whole((10, H)),
        _whole((H,)),
    ],
    out_specs=[_blocked()],
    out_shape=[jax.ShapeDtypeStruct((N, H), jnp.float32)],
    compiler_params=pltpu.CompilerParams(dimension_semantics=("parallel",)),
)


def _tables_body(h_ref, a0, a1, b0, b1_, bm1a, bm1b, wo, bo_ref,
                 ha0_o, ha1_o, hb0_o, hb1_o, lg_o):
    h = h_ref[...]
    ha0_o[...] = _dotbf(h, a0[...])
    ha1_o[...] = _dotbf(h, a1[...])
    hb0_o[...] = _dotbf(h, b0[...]) + bm1a[...]
    hb1_o[...] = _dotbf(h, b1_[...]) + bm1b[...]
    lg_o[...] = _dotbf(h, wo[...]) + bo_ref[...]


_tables = pl.pallas_call(
    _tables_body,
    grid=(NBLK,),
    in_specs=[
        _blocked(),
        _whole((H, HC)), _whole((H, HC)),
        _whole((H, HC)), _whole((H, HC)),
        _whole((HC,)), _whole((HC,)),
        _whole((H, 9)), _whole((9,)),
    ],
    out_specs=[_hc_blocked(), _hc_blocked(), _hc_blocked(), _hc_blocked(),
               pl.BlockSpec((BLK, 9), lambda i: (i, 0))],
    out_shape=[
        jax.ShapeDtypeStruct((N, HC), jnp.float32),
        jax.ShapeDtypeStruct((N, HC), jnp.float32),
        jax.ShapeDtypeStruct((N, HC), jnp.float32),
        jax.ShapeDtypeStruct((N, HC), jnp.float32),
        jax.ShapeDtypeStruct((N, 9), jnp.float32),
    ],
    compiler_params=pltpu.CompilerParams(dimension_semantics=("parallel",)),
)


def _gmm_body(h_ref, r2, deg,
              wm2a, wm2b, bm2_ref,
              wir, wiz, win_, bir, biz, bin2,
              whr, whz, whn, bhr, bhz, bhn,
              sr_o, sz_o, in_o, hn_o):
    h = h_ref[...]
    dcol = deg[...][:, :1]
    agg = _dot_exact(r2[0], wm2a[...])
    agg += _dot_exact(r2[1], wm2b[...])
    agg += dcol * bm2_ref[...]
    i_r = _dotbf(agg, wir[...]) + bir[...]
    i_z = _dotbf(agg, wiz[...]) + biz[...]
    i_n = _dotbf(agg, win_[...]) + bin2[...]
    h_r = _dotbf(h, whr[...]) + bhr[...]
    h_z = _dotbf(h, whz[...]) + bhz[...]
    h_n = _dotbf(h, whn[...]) + bhn[...]
    sr_o[...] = i_r + h_r
    sz_o[...] = i_z + h_z
    in_o[...] = i_n
    hn_o[...] = h_n


_gmm = pl.pallas_call(
    _gmm_body,
    grid=(NBLK,),
    in_specs=[
        _blocked(),
        _split_in(),
        pl.BlockSpec((BLK, 16), lambda i: (i, 0)),
        _whole((HC, H)), _whole((HC, H)), _whole((H,)),
        _whole((H, H)), _whole((H, H)), _whole((H, H)),
        _whole((H,)), _whole((H,)), _whole((H,)),
        _whole((H, H)), _whole((H, H)), _whole((H, H)),
        _whole((H,)), _whole((H,)), _whole((H,)),
    ],
    out_specs=[_blocked(), _blocked(), _blocked(), _blocked()],
    out_shape=[
        jax.ShapeDtypeStruct((N, H), jnp.float32),
        jax.ShapeDtypeStruct((N, H), jnp.float32),
        jax.ShapeDtypeStruct((N, H), jnp.float32),
        jax.ShapeDtypeStruct((N, H), jnp.float32),
    ],
    compiler_params=pltpu.CompilerParams(dimension_semantics=("parallel",)),
)


def _lnorm(t, g, b):
    m = jnp.mean(t, axis=-1, keepdims=True)
    v = jnp.var(t, axis=-1, keepdims=True)
    return (t - m) / jnp.sqrt(v + LN_EPS) * g + b


def _pad_cols(w, total=2 * HC):
    return jnp.pad(w, ((0, 0), (0, total - w.shape[1])))


def kernel(x, edge_index, W_in, b_in, ln1_g, ln1_b, pos, Wm1, bm1, Wm2, bm2,
           Wih, bih, Whh, bhh, ln2_g, ln2_b, Wo, bo):
    n_graphs = N // 81
    srcr = edge_index[1].reshape(NSUB, NGRP, G, K)
    dstr = edge_index[0].reshape(NSUB, NGRP, G, K)
    a_p = _pad_cols(Wm1[:H])
    b_p = _pad_cols(Wm1[H:])
    a0, a1 = a_p[:, :HC], a_p[:, HC:]
    b0, b1_ = b_p[:, :HC], b_p[:, HC:]
    bm1_p = jnp.pad(bm1, (0, 2 * HC - H))
    bm1a, bm1b = bm1_p[:HC], bm1_p[HC:]
    wm2_r = Wm2.astype(jnp.bfloat16).astype(jnp.float32)
    wm2a = wm2_r[:HC]
    wm2b = jnp.pad(wm2_r[HC:], ((0, 2 * HC - H), (0, 0)))
    wir, wiz, win_ = Wih[:, :H], Wih[:, H:2 * H], Wih[:, 2 * H:]
    bir, biz, bin2 = bih[:H], bih[H:2 * H], bih[2 * H:]
    whr, whz, whn = Whh[:, :H], Whh[:, H:2 * H], Whh[:, 2 * H:]
    bhr, bhz, bhn = bhh[:H], bhh[H:2 * H], bhh[2 * H:]

    ones_t = jnp.ones((N, HC), jnp.float32)
    zeros_t = jnp.zeros((N, HC), jnp.float32)
    deg = _edge_pass(ones_t, ones_t, zeros_t, zeros_t, srcr, dstr)[0, :, :16]

    (t0,) = _matin(x, W_in, b_in)
    h = _lnorm(t0, ln1_g, ln1_b) + jnp.tile(pos, (n_graphs, 1))
    ha0, ha1, hb0, hb1, _ = _tables(h, a0, a1, b0, b1_, bm1a, bm1b, Wo, bo)
    logits = []
    for _ in range(NSTEPS):
        r2 = _edge_pass(ha0, ha1, hb0, hb1, srcr, dstr)
        s_r, s_z, i_n, h_n = _gmm(h, r2, deg, wm2a, wm2b, bm2,
                                  wir, wiz, win_, bir, biz, bin2,
                                  whr, whz, whn, bhr, bhz, bhn)
        r = jax.nn.sigmoid(s_r)
        z = jax.nn.sigmoid(s_z)
        n = jnp.tanh(i_n + r * h_n)
        h = _lnorm((1.0 - z) * n + z * h, ln2_g, ln2_b)
        ha0, ha1, hb0, hb1, lg = _tables(h, a0, a1, b0, b1_, bm1a, bm1b,
                                         Wo, bo)
        logits.append(lg)
    return logits[-1], jnp.stack(logits)


# speedup vs baseline: 3.7426x; 1.1589x over previous
import functools

import jax
import jax.numpy as jnp
from jax import lax
from jax.experimental import pallas as pl
from jax.experimental.pallas import tpu as pltpu
from jax.experimental.pallas import tpu_sc as plsc

N = 10368
E = 207360
H = 192
HC = 128
NSTEPS = 4
NCORES = 2
NSUB = 16
EPT = E // NSUB
K = 96
NCHUNK = EPT // K
G = 45
NGRP = NCHUNK // G
RPT = N // NSUB
NBLK = 8
BLK = N // NBLK
LN_EPS = 1e-5

_sc_mesh = plsc.VectorSubcoreMesh(core_axis_name="c", subcore_axis_name="s")


@functools.partial(
    pl.kernel,
    out_type=jax.ShapeDtypeStruct((NCORES, N, HC), jnp.float32),
    mesh=_sc_mesh,
    scratch_types=[
        pltpu.VMEM((G, K), jnp.int32),
        pltpu.VMEM((G, K), jnp.int32),
        pltpu.VMEM((K, HC), jnp.float32),
        pltpu.VMEM((K, HC), jnp.float32),
        pltpu.VMEM_SHARED((N, HC), jnp.float32),
        pltpu.SemaphoreType.DMA,
        pltpu.SemaphoreType.DMA,
    ],
)
def _edge_pass(ha0, ha1, hb0, hb1, srcr, dstr, out, si, di, ar, br, acc,
               s1, s2):
    cid = lax.axis_index("c")
    sid = lax.axis_index("s")
    row0 = sid * RPT

    zero16 = jnp.zeros((16,), jnp.float32)

    def zrow(i, carry):
        r = i // (HC // 16)
        c = (i % (HC // 16)) * 16
        ar[r, pl.ds(c, 16)] = zero16
        return carry

    lax.fori_loop(0, K * (HC // 16), zrow, 0)
    for j in range(RPT // K):
        pltpu.sync_copy(ar, acc.at[pl.ds(row0 + j * K, K)])
    rem = RPT - (RPT // K) * K
    if rem:
        pltpu.sync_copy(ar.at[pl.ds(0, rem)],
                        acc.at[pl.ds(row0 + (RPT // K) * K, rem)])
    plsc.subcore_barrier()

    def chunk(c, carry):
        @pl.when(cid == 0)
        def _():
            d1 = pltpu.async_copy(ha0.at[si.at[c]], ar, s1)
            d2 = pltpu.async_copy(hb0.at[di.at[c]], br, s2)
            d1.wait()
            d2.wait()

        @pl.when(cid == 1)
        def _():
            d1 = pltpu.async_copy(ha1.at[si.at[c]], ar, s1)
            d2 = pltpu.async_copy(hb1.at[di.at[c]], br, s2)
            d1.wait()
            d2.wait()

        def rowfn(r, carry2):
            for cc in range(HC // 16):
                a = ar[r, pl.ds(cc * 16, 16)]
                b = br[r, pl.ds(cc * 16, 16)]
                v = jnp.maximum(a + b, 0.0)
                u = lax.bitcast_convert_type(v, jnp.int32)
                u = u & jnp.int32(-65536)
                ar[r, pl.ds(cc * 16, 16)] = lax.bitcast_convert_type(
                    u, jnp.float32)
            return carry2

        lax.fori_loop(0, K, rowfn, 0)
        pltpu.sync_copy(ar, acc.at[di.at[c]], add=True)
        return carry

    for g in range(NGRP):
        pltpu.sync_copy(srcr.at[sid, g], si)
        pltpu.sync_copy(dstr.at[sid, g], di)
        lax.fori_loop(0, G, chunk, 0)
    plsc.subcore_barrier()
    pltpu.sync_copy(acc.at[pl.ds(row0, RPT)], out.at[cid, pl.ds(row0, RPT)])


@functools.partial(
    pl.kernel,
    out_type=jax.ShapeDtypeStruct((NCORES, N, HC), jnp.float32),
    mesh=_sc_mesh,
    scratch_types=[
        pltpu.VMEM((G, K), jnp.int32),
        pltpu.VMEM((K, HC), jnp.float32),
        pltpu.VMEM_SHARED((N, HC), jnp.float32),
    ],
)
def _deg_pass(dstr, out, di, ones_b, acc):
    cid = lax.axis_index("c")
    sid = lax.axis_index("s")
    row0 = sid * RPT

    one16 = jnp.ones((16,), jnp.float32)
    zero16 = jnp.zeros((16,), jnp.float32)

    def zrow(i, carry):
        r = i // (HC // 16)
        c = (i % (HC // 16)) * 16
        ones_b[r, pl.ds(c, 16)] = zero16
        return carry

    lax.fori_loop(0, K * (HC // 16), zrow, 0)
    for j in range(RPT // K):
        pltpu.sync_copy(ones_b, acc.at[pl.ds(row0 + j * K, K)])
    rem = RPT - (RPT // K) * K
    if rem:
        pltpu.sync_copy(ones_b.at[pl.ds(0, rem)],
                        acc.at[pl.ds(row0 + (RPT // K) * K, rem)])

    def orow(i, carry):
        r = i // (HC // 16)
        c = (i % (HC // 16)) * 16
        ones_b[r, pl.ds(c, 16)] = one16
        return carry

    lax.fori_loop(0, K * (HC // 16), orow, 0)
    plsc.subcore_barrier()

    def chunk(c, carry):
        pltpu.sync_copy(ones_b, acc.at[di.at[c]], add=True)
        return carry

    for g in range(NGRP):
        pltpu.sync_copy(dstr.at[sid, g], di)
        lax.fori_loop(0, G, chunk, 0)
    plsc.subcore_barrier()
    pltpu.sync_copy(acc.at[pl.ds(row0, RPT)], out.at[cid, pl.ds(row0, RPT)])


def _whole(shape):
    nd = len(shape)
    return pl.BlockSpec(shape, lambda i, _n=nd: (0,) * _n)


def _blocked():
    return pl.BlockSpec((BLK, H), lambda i: (i, 0))


def _split_in():
    return pl.BlockSpec((NCORES, BLK, HC), lambda i: (0, i, 0))


def _hc_blocked():
    return pl.BlockSpec((BLK, HC), lambda i: (i, 0))


def _dotbf(a, b):
    return jnp.dot(a.astype(jnp.bfloat16), b.astype(jnp.bfloat16),
                   preferred_element_type=jnp.float32)


def _dot_exact(a, b):
    a = a.astype(jnp.float32)
    hi = a.astype(jnp.bfloat16)
    mid = (a - hi.astype(jnp.float32)).astype(jnp.bfloat16)
    lo = (a - hi.astype(jnp.float32)
          - mid.astype(jnp.float32)).astype(jnp.bfloat16)
    bb = b.astype(jnp.bfloat16)
    f32 = jnp.float32
    return (jnp.dot(hi, bb, preferred_element_type=f32)
            + jnp.dot(mid, bb, preferred_element_type=f32)
            + jnp.dot(lo, bb, preferred_element_type=f32))


def _matin_body(x_ref, win, bin_, t_o):
    t_o[...] = _dotbf(x_ref[...], win[...]) + bin_[...]


_matin = pl.pallas_call(
    _matin_body,
    grid=(NBLK,),
    in_specs=[
        pl.BlockSpec((BLK, 10), lambda i: (i, 0)),
        _whole((10, H)),
        _whole((H,)),
    ],
    out_specs=[_blocked()],
    out_shape=[jax.ShapeDtypeStruct((N, H), jnp.float32)],
    compiler_params=pltpu.CompilerParams(dimension_semantics=("parallel",)),
)


def _tables_body(h_ref, a0, a1, b0, b1_, bm1a, bm1b, wo, bo_ref,
                 ha0_o, ha1_o, hb0_o, hb1_o, lg_o):
    h = h_ref[...]
    ha0_o[...] = _dotbf(h, a0[...])
    ha1_o[...] = _dotbf(h, a1[...])
    hb0_o[...] = _dotbf(h, b0[...]) + bm1a[...]
    hb1_o[...] = _dotbf(h, b1_[...]) + bm1b[...]
    lg_o[...] = _dotbf(h, wo[...]) + bo_ref[...]


_tables = pl.pallas_call(
    _tables_body,
    grid=(NBLK,),
    in_specs=[
        _blocked(),
        _whole((H, HC)), _whole((H, HC)),
        _whole((H, HC)), _whole((H, HC)),
        _whole((HC,)), _whole((HC,)),
        _whole((H, 9)), _whole((9,)),
    ],
    out_specs=[_hc_blocked(), _hc_blocked(), _hc_blocked(), _hc_blocked(),
               pl.BlockSpec((BLK, 9), lambda i: (i, 0))],
    out_shape=[
        jax.ShapeDtypeStruct((N, HC), jnp.float32),
        jax.ShapeDtypeStruct((N, HC), jnp.float32),
        jax.ShapeDtypeStruct((N, HC), jnp.float32),
        jax.ShapeDtypeStruct((N, HC), jnp.float32),
        jax.ShapeDtypeStruct((N, 9), jnp.float32),
    ],
    compiler_params=pltpu.CompilerParams(dimension_semantics=("parallel",)),
)


def _gmm_body(h_ref, r2, deg,
              wm2a, wm2b, bm2_ref,
              wir, wiz, win_, bir, biz, bin2,
              whr, whz, whn, bhr, bhz, bhn,
              sr_o, sz_o, in_o, hn_o):
    h = h_ref[...]
    dcol = deg[...][:, :1]
    agg = _dot_exact(r2[0], wm2a[...])
    agg += _dot_exact(r2[1], wm2b[...])
    agg += dcol * bm2_ref[...]
    i_r = _dotbf(agg, wir[...]) + bir[...]
    i_z = _dotbf(agg, wiz[...]) + biz[...]
    i_n = _dotbf(agg, win_[...]) + bin2[...]
    h_r = _dotbf(h, whr[...]) + bhr[...]
    h_z = _dotbf(h, whz[...]) + bhz[...]
    h_n = _dotbf(h, whn[...]) + bhn[...]
    sr_o[...] = i_r + h_r
    sz_o[...] = i_z + h_z
    in_o[...] = i_n
    hn_o[...] = h_n


_gmm = pl.pallas_call(
    _gmm_body,
    grid=(NBLK,),
    in_specs=[
        _blocked(),
        _split_in(),
        pl.BlockSpec((BLK, 16), lambda i: (i, 0)),
        _whole((HC, H)), _whole((HC, H)), _whole((H,)),
        _whole((H, H)), _whole((H, H)), _whole((H, H)),
        _whole((H,)), _whole((H,)), _whole((H,)),
        _whole((H, H)), _whole((H, H)), _whole((H, H)),
        _whole((H,)), _whole((H,)), _whole((H,)),
    ],
    out_specs=[_blocked(), _blocked(), _blocked(), _blocked()],
    out_shape=[
        jax.ShapeDtypeStruct((N, H), jnp.float32),
        jax.ShapeDtypeStruct((N, H), jnp.float32),
        jax.ShapeDtypeStruct((N, H), jnp.float32),
        jax.ShapeDtypeStruct((N, H), jnp.float32),
    ],
    compiler_params=pltpu.CompilerParams(dimension_semantics=("parallel",)),
)


def _lnorm(t, g, b):
    m = jnp.mean(t, axis=-1, keepdims=True)
    v = jnp.var(t, axis=-1, keepdims=True)
    return (t - m) / jnp.sqrt(v + LN_EPS) * g + b


def _pad_cols(w, total=2 * HC):
    return jnp.pad(w, ((0, 0), (0, total - w.shape[1])))


def kernel(x, edge_index, W_in, b_in, ln1_g, ln1_b, pos, Wm1, bm1, Wm2, bm2,
           Wih, bih, Whh, bhh, ln2_g, ln2_b, Wo, bo):
    n_graphs = N // 81
    srcr = edge_index[1].reshape(NSUB, NGRP, G, K)
    dstr = edge_index[0].reshape(NSUB, NGRP, G, K)
    a_p = _pad_cols(Wm1[:H])
    b_p = _pad_cols(Wm1[H:])
    a0, a1 = a_p[:, :HC], a_p[:, HC:]
    b0, b1_ = b_p[:, :HC], b_p[:, HC:]
    bm1_p = jnp.pad(bm1, (0, 2 * HC - H))
    bm1a, bm1b = bm1_p[:HC], bm1_p[HC:]
    wm2_r = Wm2.astype(jnp.bfloat16).astype(jnp.float32)
    wm2a = wm2_r[:HC]
    wm2b = jnp.pad(wm2_r[HC:], ((0, 2 * HC - H), (0, 0)))
    wir, wiz, win_ = Wih[:, :H], Wih[:, H:2 * H], Wih[:, 2 * H:]
    bir, biz, bin2 = bih[:H], bih[H:2 * H], bih[2 * H:]
    whr, whz, whn = Whh[:, :H], Whh[:, H:2 * H], Whh[:, 2 * H:]
    bhr, bhz, bhn = bhh[:H], bhh[H:2 * H], bhh[2 * H:]

    deg = _deg_pass(dstr)[0, :, :16]

    (t0,) = _matin(x, W_in, b_in)
    h = _lnorm(t0, ln1_g, ln1_b) + jnp.tile(pos, (n_graphs, 1))
    ha0, ha1, hb0, hb1, _ = _tables(h, a0, a1, b0, b1_, bm1a, bm1b, Wo, bo)
    logits = []
    for _ in range(NSTEPS):
        r2 = _edge_pass(ha0, ha1, hb0, hb1, srcr, dstr)
        s_r, s_z, i_n, h_n = _gmm(h, r2, deg, wm2a, wm2b, bm2,
                                  wir, wiz, win_, bir, biz, bin2,
                                  whr, whz, whn, bhr, bhz, bhn)
        r = jax.nn.sigmoid(s_r)
        z = jax.nn.sigmoid(s_z)
        n = jnp.tanh(i_n + r * h_n)
        h = _lnorm((1.0 - z) * n + z * h, ln2_g, ln2_b)
        ha0, ha1, hb0, hb1, lg = _tables(h, a0, a1, b0, b1_, bm1a, bm1b,
                                         Wo, bo)
        logits.append(lg)
    return logits[-1], jnp.stack(logits)


# speedup vs baseline: 4.4506x; 1.1892x over previous
import functools

import jax
import jax.numpy as jnp
from jax import lax
from jax.experimental import pallas as pl
from jax.experimental.pallas import tpu as pltpu
from jax.experimental.pallas import tpu_sc as plsc

N = 10368
E = 207360
H = 192
HC = 128
NSTEPS = 4
NCORES = 2
NSUB = 16
EPT = E // NSUB
K = 40
NCHUNK = EPT // K
G = 36
NGRP = NCHUNK // G
KD = 96
GD = 45
NGD = (EPT // KD) // GD
RPT = N // NSUB
NBLK = 8
BLK = N // NBLK
LN_EPS = 1e-5

_sc_mesh = plsc.VectorSubcoreMesh(core_axis_name="c", subcore_axis_name="s")


@functools.partial(
    pl.kernel,
    out_type=jax.ShapeDtypeStruct((NCORES, N, HC), jnp.float32),
    mesh=_sc_mesh,
    scratch_types=[
        pltpu.VMEM((2, G, K), jnp.int32),
        pltpu.VMEM((2, G, K), jnp.int32),
        pltpu.VMEM((2, K, HC), jnp.float32),
        pltpu.VMEM((2, K, HC), jnp.float32),
        pltpu.VMEM_SHARED((N, HC), jnp.float32),
        pltpu.SemaphoreType.DMA, pltpu.SemaphoreType.DMA,
        pltpu.SemaphoreType.DMA, pltpu.SemaphoreType.DMA,
        pltpu.SemaphoreType.DMA, pltpu.SemaphoreType.DMA,
    ],
)
def _edge_pass(ha0, ha1, hb0, hb1, srcr, dstr, out, si, di, ar, br, acc,
               sga0, sga1, sgb0, sgb1, ssc0, ssc1):
    cid = lax.axis_index("c")
    sid = lax.axis_index("s")
    row0 = sid * RPT
    sga = (sga0, sga1)
    sgb = (sgb0, sgb1)
    ssc = (ssc0, ssc1)
    zero16 = jnp.zeros((16,), jnp.float32)

    for p in range(2):
        def zrow(i, carry, _p=p):
            r = i // (HC // 16)
            c = (i % (HC // 16)) * 16
            ar[_p, r, pl.ds(c, 16)] = zero16
            return carry
        lax.fori_loop(0, K * (HC // 16), zrow, 0)
    for j in range(RPT // K):
        pltpu.sync_copy(ar.at[0], acc.at[pl.ds(row0 + j * K, K)])
    rem = RPT - (RPT // K) * K
    if rem:
        pltpu.sync_copy(ar.at[0, pl.ds(0, rem)],
                        acc.at[pl.ds(row0 + (RPT // K) * K, rem)])
    plsc.subcore_barrier()

    pltpu.sync_copy(srcr.at[sid, 0], si.at[0])
    pltpu.sync_copy(dstr.at[sid, 0], di.at[0])
    def gathers(gp, c, p):
        @pl.when(cid == 0)
        def _():
            pltpu.async_copy(ha0.at[si.at[gp, c]], ar.at[p], sga[p])
            pltpu.async_copy(hb0.at[di.at[gp, c]], br.at[p], sgb[p])

        @pl.when(cid == 1)
        def _():
            pltpu.async_copy(ha1.at[si.at[gp, c]], ar.at[p], sga[p])
            pltpu.async_copy(hb1.at[di.at[gp, c]], br.at[p], sgb[p])

    def wait_gathers(p):
        pltpu.make_async_copy(ha0.at[pl.ds(0, K)], ar.at[p], sga[p]).wait()
        pltpu.make_async_copy(hb0.at[pl.ds(0, K)], br.at[p], sgb[p]).wait()

    def drain_scatter(p):
        pltpu.make_async_copy(ar.at[p], acc.at[pl.ds(row0, K)], ssc[p]).wait()

    def compute(p):
        def rowfn(r, carry):
            for cc in range(HC // 16):
                a = ar[p, r, pl.ds(cc * 16, 16)]
                b = br[p, r, pl.ds(cc * 16, 16)]
                v = jnp.maximum(a + b, 0.0)
                u = lax.bitcast_convert_type(v, jnp.int32)
                u = u + jnp.int32(0x7FFF) + (lax.shift_right_logical(u, 16)
                                             & jnp.int32(1))
                u = u & jnp.int32(-65536)
                ar[p, r, pl.ds(cc * 16, 16)] = lax.bitcast_convert_type(
                    u, jnp.float32)
            return carry
        lax.fori_loop(0, K, rowfn, 0)

    gathers(0, 0, 0)
    for g in range(NGRP):
        gp = g % 2
        if g + 1 < NGRP:
            pltpu.sync_copy(srcr.at[sid, g + 1], si.at[1 - gp])
            pltpu.sync_copy(dstr.at[sid, g + 1], di.at[1 - gp])

        def pair(t2, carry, _gp=gp, _g=g):
            for p in range(2):
                c = 2 * t2 + p
                wait_gathers(p)
                if _g == 0 and p == 0:
                    @pl.when(t2 > 0)
                    def _():
                        drain_scatter(1)
                else:
                    drain_scatter(1 - p)
                if p == 0:
                    gathers(_gp, c + 1, 1)
                else:
                    @pl.when(t2 < G // 2 - 1)
                    def _():
                        gathers(_gp, c + 1, 0)
                compute(p)
                pltpu.async_copy(ar.at[p], acc.at[di.at[_gp, c]], ssc[p],
                                 add=True)
            return carry

        lax.fori_loop(0, G // 2, pair, 0)
        if g + 1 < NGRP:
            gathers(1 - gp, 0, 0)
    drain_scatter(1)
    plsc.subcore_barrier()
    pltpu.sync_copy(acc.at[pl.ds(row0, RPT)], out.at[cid, pl.ds(row0, RPT)])


@functools.partial(
    pl.kernel,
    out_type=jax.ShapeDtypeStruct((NCORES, N, HC), jnp.float32),
    mesh=_sc_mesh,
    scratch_types=[
        pltpu.VMEM((GD, KD), jnp.int32),
        pltpu.VMEM((KD, HC), jnp.float32),
        pltpu.VMEM_SHARED((N, HC), jnp.float32),
    ],
)
def _deg_pass(dstr, out, di, ones_b, acc):
    cid = lax.axis_index("c")
    sid = lax.axis_index("s")
    row0 = sid * RPT

    one16 = jnp.ones((16,), jnp.float32)
    zero16 = jnp.zeros((16,), jnp.float32)

    def zrow(i, carry):
        r = i // (HC // 16)
        c = (i % (HC // 16)) * 16
        ones_b[r, pl.ds(c, 16)] = zero16
        return carry

    lax.fori_loop(0, KD * (HC // 16), zrow, 0)
    for j in range(RPT // KD):
        pltpu.sync_copy(ones_b, acc.at[pl.ds(row0 + j * KD, KD)])
    rem = RPT - (RPT // KD) * K
    if rem:
        pltpu.sync_copy(ones_b.at[pl.ds(0, rem)],
                        acc.at[pl.ds(row0 + (RPT // KD) * KD, rem)])

    def orow(i, carry):
        r = i // (HC // 16)
        c = (i % (HC // 16)) * 16
        ones_b[r, pl.ds(c, 16)] = one16
        return carry

    lax.fori_loop(0, KD * (HC // 16), orow, 0)
    plsc.subcore_barrier()

    def chunk(c, carry):
        pltpu.sync_copy(ones_b, acc.at[di.at[c]], add=True)
        return carry

    for g in range(NGD):
        pltpu.sync_copy(dstr.at[sid, g], di)
        lax.fori_loop(0, GD, chunk, 0)
    plsc.subcore_barrier()
    pltpu.sync_copy(acc.at[pl.ds(row0, RPT)], out.at[cid, pl.ds(row0, RPT)])


def _whole(shape):
    nd = len(shape)
    return pl.BlockSpec(shape, lambda i, _n=nd: (0,) * _n)


def _blocked():
    return pl.BlockSpec((BLK, H), lambda i: (i, 0))


def _split_in():
    return pl.BlockSpec((NCORES, BLK, HC), lambda i: (0, i, 0))


def _hc_blocked():
    return pl.BlockSpec((BLK, HC), lambda i: (i, 0))


def _dotbf(a, b):
    return jnp.dot(a.astype(jnp.bfloat16), b.astype(jnp.bfloat16),
                   preferred_element_type=jnp.float32)


def _dot_exact(a, b):
    a = a.astype(jnp.float32)
    hi = a.astype(jnp.bfloat16)
    mid = (a - hi.astype(jnp.float32)).astype(jnp.bfloat16)
    lo = (a - hi.astype(jnp.float32)
          - mid.astype(jnp.float32)).astype(jnp.bfloat16)
    bb = b.astype(jnp.bfloat16)
    f32 = jnp.float32
    return (jnp.dot(hi, bb, preferred_element_type=f32)
            + jnp.dot(mid, bb, preferred_element_type=f32)
            + jnp.dot(lo, bb, preferred_element_type=f32))


def _matin_body(x_ref, win, bin_, t_o):
    t_o[...] = _dotbf(x_ref[...], win[...]) + bin_[...]


_matin = pl.pallas_call(
    _matin_body,
    grid=(NBLK,),
    in_specs=[
        pl.BlockSpec((BLK, 10), lambda i: (i, 0)),
        _whole((10, H)),
        _whole((H,)),
    ],
    out_specs=[_blocked()],
    out_shape=[jax.ShapeDtypeStruct((N, H), jnp.float32)],
    compiler_params=pltpu.CompilerParams(dimension_semantics=("parallel",)),
)


def _tables_body(h_ref, a0, a1, b0, b1_, bm1a, bm1b, wo, bo_ref,
                 ha0_o, ha1_o, hb0_o, hb1_o, lg_o):
    h = h_ref[...]
    ha0_o[...] = _dotbf(h, a0[...])
    ha1_o[...] = _dotbf(h, a1[...])
    hb0_o[...] = _dotbf(h, b0[...]) + bm1a[...]
    hb1_o[...] = _dotbf(h, b1_[...]) + bm1b[...]
    lg_o[...] = _dotbf(h, wo[...]) + bo_ref[...]


_tables = pl.pallas_call(
    _tables_body,
    grid=(NBLK,),
    in_specs=[
        _blocked(),
        _whole((H, HC)), _whole((H, HC)),
        _whole((H, HC)), _whole((H, HC)),
        _whole((HC,)), _whole((HC,)),
        _whole((H, 9)), _whole((9,)),
    ],
    out_specs=[_hc_blocked(), _hc_blocked(), _hc_blocked(), _hc_blocked(),
               pl.BlockSpec((BLK, 9), lambda i: (i, 0))],
    out_shape=[
        jax.ShapeDtypeStruct((N, HC), jnp.float32),
        jax.ShapeDtypeStruct((N, HC), jnp.float32),
        jax.ShapeDtypeStruct((N, HC), jnp.float32),
        jax.ShapeDtypeStruct((N, HC), jnp.float32),
        jax.ShapeDtypeStruct((N, 9), jnp.float32),
    ],
    compiler_params=pltpu.CompilerParams(dimension_semantics=("parallel",)),
)


def _gmm_body(h_ref, r2, deg,
              wm2a, wm2b, bm2_ref,
              wir, wiz, win_, bir, biz, bin2,
              whr, whz, whn, bhr, bhz, bhn,
              sr_o, sz_o, in_o, hn_o):
    h = h_ref[...]
    dcol = deg[...][:, :1]
    agg = _dot_exact(r2[0], wm2a[...])
    agg += _dot_exact(r2[1], wm2b[...])
    agg += dcol * bm2_ref[...]
    i_r = _dotbf(agg, wir[...]) + bir[...]
    i_z = _dotbf(agg, wiz[...]) + biz[...]
    i_n = _dotbf(agg, win_[...]) + bin2[...]
    h_r = _dotbf(h, whr[...]) + bhr[...]
    h_z = _dotbf(h, whz[...]) + bhz[...]
    h_n = _dotbf(h, whn[...]) + bhn[...]
    sr_o[...] = i_r + h_r
    sz_o[...] = i_z + h_z
    in_o[...] = i_n
    hn_o[...] = h_n


_gmm = pl.pallas_call(
    _gmm_body,
    grid=(NBLK,),
    in_specs=[
        _blocked(),
        _split_in(),
        pl.BlockSpec((BLK, 16), lambda i: (i, 0)),
        _whole((HC, H)), _whole((HC, H)), _whole((H,)),
        _whole((H, H)), _whole((H, H)), _whole((H, H)),
        _whole((H,)), _whole((H,)), _whole((H,)),
        _whole((H, H)), _whole((H, H)), _whole((H, H)),
        _whole((H,)), _whole((H,)), _whole((H,)),
    ],
    out_specs=[_blocked(), _blocked(), _blocked(), _blocked()],
    out_shape=[
        jax.ShapeDtypeStruct((N, H), jnp.float32),
        jax.ShapeDtypeStruct((N, H), jnp.float32),
        jax.ShapeDtypeStruct((N, H), jnp.float32),
        jax.ShapeDtypeStruct((N, H), jnp.float32),
    ],
    compiler_params=pltpu.CompilerParams(dimension_semantics=("parallel",)),
)


def _lnorm(t, g, b):
    m = jnp.mean(t, axis=-1, keepdims=True)
    v = jnp.var(t, axis=-1, keepdims=True)
    return (t - m) / jnp.sqrt(v + LN_EPS) * g + b


def _pad_cols(w, total=2 * HC):
    return jnp.pad(w, ((0, 0), (0, total - w.shape[1])))


def kernel(x, edge_index, W_in, b_in, ln1_g, ln1_b, pos, Wm1, bm1, Wm2, bm2,
           Wih, bih, Whh, bhh, ln2_g, ln2_b, Wo, bo):
    n_graphs = N // 81
    srcr = edge_index[1].reshape(NSUB, NGRP, G, K)
    dstr = edge_index[0].reshape(NSUB, NGRP, G, K)
    dstr_d = edge_index[0].reshape(NSUB, NGD, GD, KD)
    a_p = _pad_cols(Wm1[:H])
    b_p = _pad_cols(Wm1[H:])
    a0, a1 = a_p[:, :HC], a_p[:, HC:]
    b0, b1_ = b_p[:, :HC], b_p[:, HC:]
    bm1_p = jnp.pad(bm1, (0, 2 * HC - H))
    bm1a, bm1b = bm1_p[:HC], bm1_p[HC:]
    wm2_r = Wm2.astype(jnp.bfloat16).astype(jnp.float32)
    wm2a = wm2_r[:HC]
    wm2b = jnp.pad(wm2_r[HC:], ((0, 2 * HC - H), (0, 0)))
    wir, wiz, win_ = Wih[:, :H], Wih[:, H:2 * H], Wih[:, 2 * H:]
    bir, biz, bin2 = bih[:H], bih[H:2 * H], bih[2 * H:]
    whr, whz, whn = Whh[:, :H], Whh[:, H:2 * H], Whh[:, 2 * H:]
    bhr, bhz, bhn = bhh[:H], bhh[H:2 * H], bhh[2 * H:]

    deg = _deg_pass(dstr_d)[0, :, :16]

    (t0,) = _matin(x, W_in, b_in)
    h = _lnorm(t0, ln1_g, ln1_b) + jnp.tile(pos, (n_graphs, 1))
    ha0, ha1, hb0, hb1, _ = _tables(h, a0, a1, b0, b1_, bm1a, bm1b, Wo, bo)
    logits = []
    for _ in range(NSTEPS):
        r2 = _edge_pass(ha0, ha1, hb0, hb1, srcr, dstr)
        s_r, s_z, i_n, h_n = _gmm(h, r2, deg, wm2a, wm2b, bm2,
                                  wir, wiz, win_, bir, biz, bin2,
                                  whr, whz, whn, bhr, bhz, bhn)
        r = jax.nn.sigmoid(s_r)
        z = jax.nn.sigmoid(s_z)
        n = jnp.tanh(i_n + r * h_n)
        h = _lnorm((1.0 - z) * n + z * h, ln2_g, ln2_b)
        ha0, ha1, hb0, hb1, lg = _tables(h, a0, a1, b0, b1_, bm1a, bm1b,
                                         Wo, bo)
        logits.append(lg)
    return logits[-1], jnp.stack(logits)


# speedup vs baseline: 4.9148x; 1.1043x over previous
import functools

import jax
import jax.numpy as jnp
from jax import lax
from jax.experimental import pallas as pl
from jax.experimental.pallas import tpu as pltpu
from jax.experimental.pallas import tpu_sc as plsc

N = 10368
E = 207360
H = 192
HC = 128
NSTEPS = 4
NCORES = 2
NSUB = 16
EPT = E // NSUB
K = 40
NCHUNK = EPT // K
G = 36
NGRP = NCHUNK // G
KD = 96
GD = 45
NGD = (EPT // KD) // GD
RPT = N // NSUB
NBLK = 8
BLK = N // NBLK
LN_EPS = 1e-5

_sc_mesh = plsc.VectorSubcoreMesh(core_axis_name="c", subcore_axis_name="s")


@functools.partial(
    pl.kernel,
    out_type=jax.ShapeDtypeStruct((NCORES, N, HC), jnp.float32),
    mesh=_sc_mesh,
    scratch_types=[
        pltpu.VMEM((2, G, K), jnp.int32),
        pltpu.VMEM((2, G, K), jnp.int32),
        pltpu.VMEM((2, K, HC), jnp.float32),
        pltpu.VMEM((2, K, HC), jnp.float32),
        pltpu.VMEM_SHARED((N, HC), jnp.float32),
        pltpu.SemaphoreType.DMA, pltpu.SemaphoreType.DMA,
        pltpu.SemaphoreType.DMA, pltpu.SemaphoreType.DMA,
        pltpu.SemaphoreType.DMA, pltpu.SemaphoreType.DMA,
    ],
)
def _edge_pass(ha0, ha1, hb0, hb1, srcr, dstr, out, si, di, ar, br, acc,
               sga0, sga1, sgb0, sgb1, ssc0, ssc1):
    cid = lax.axis_index("c")
    sid = lax.axis_index("s")
    row0 = sid * RPT
    sga = (sga0, sga1)
    sgb = (sgb0, sgb1)
    ssc = (ssc0, ssc1)
    zero16 = jnp.zeros((16,), jnp.float32)

    for p in range(2):
        def zrow(i, carry, _p=p):
            r = i // (HC // 16)
            c = (i % (HC // 16)) * 16
            ar[_p, r, pl.ds(c, 16)] = zero16
            return carry
        lax.fori_loop(0, K * (HC // 16), zrow, 0)
    for j in range(RPT // K):
        pltpu.sync_copy(ar.at[0], acc.at[pl.ds(row0 + j * K, K)])
    rem = RPT - (RPT // K) * K
    if rem:
        pltpu.sync_copy(ar.at[0, pl.ds(0, rem)],
                        acc.at[pl.ds(row0 + (RPT // K) * K, rem)])
    plsc.subcore_barrier()

    pltpu.sync_copy(srcr.at[sid, 0], si.at[0])
    pltpu.sync_copy(dstr.at[sid, 0], di.at[0])
    def gathers(gp, c, p):
        @pl.when(cid == 0)
        def _():
            pltpu.async_copy(ha0.at[si.at[gp, c]], ar.at[p], sga[p])
            pltpu.async_copy(hb0.at[di.at[gp, c]], br.at[p], sgb[p])

        @pl.when(cid == 1)
        def _():
            pltpu.async_copy(ha1.at[si.at[gp, c]], ar.at[p], sga[p])
            pltpu.async_copy(hb1.at[di.at[gp, c]], br.at[p], sgb[p])

    def wait_gathers(p):
        pltpu.make_async_copy(ha0.at[pl.ds(0, K)], ar.at[p], sga[p]).wait()
        pltpu.make_async_copy(hb0.at[pl.ds(0, K)], br.at[p], sgb[p]).wait()

    def drain_scatter(p):
        pltpu.make_async_copy(ar.at[p], acc.at[pl.ds(row0, K)], ssc[p]).wait()

    def compute(p):
        def rowfn(r, carry):
            for cc in range(HC // 16):
                a = ar[p, r, pl.ds(cc * 16, 16)]
                b = br[p, r, pl.ds(cc * 16, 16)]
                v = jnp.maximum(a + b, 0.0)
                u = lax.bitcast_convert_type(v, jnp.int32)
                u = u + jnp.int32(0x7FFF) + (lax.shift_right_logical(u, 16)
                                             & jnp.int32(1))
                u = u & jnp.int32(-65536)
                ar[p, r, pl.ds(cc * 16, 16)] = lax.bitcast_convert_type(
                    u, jnp.float32)
            return carry
        lax.fori_loop(0, K, rowfn, 0)

    gathers(0, 0, 0)
    for g in range(NGRP):
        gp = g % 2
        if g + 1 < NGRP:
            pltpu.sync_copy(srcr.at[sid, g + 1], si.at[1 - gp])
            pltpu.sync_copy(dstr.at[sid, g + 1], di.at[1 - gp])

        def pair(t2, carry, _gp=gp, _g=g):
            for p in range(2):
                c = 2 * t2 + p
                wait_gathers(p)
                if _g == 0 and p == 0:
                    @pl.when(t2 > 0)
                    def _():
                        drain_scatter(1)
                else:
                    drain_scatter(1 - p)
                if p == 0:
                    gathers(_gp, c + 1, 1)
                else:
                    @pl.when(t2 < G // 2 - 1)
                    def _():
                        gathers(_gp, c + 1, 0)
                compute(p)
                pltpu.async_copy(ar.at[p], acc.at[di.at[_gp, c]], ssc[p],
                                 add=True)
            return carry

        lax.fori_loop(0, G // 2, pair, 0)
        if g + 1 < NGRP:
            gathers(1 - gp, 0, 0)
    drain_scatter(1)
    plsc.subcore_barrier()
    pltpu.sync_copy(acc.at[pl.ds(row0, RPT)], out.at[cid, pl.ds(row0, RPT)])


@functools.partial(
    pl.kernel,
    out_type=jax.ShapeDtypeStruct((NCORES, N, HC), jnp.float32),
    mesh=_sc_mesh,
    scratch_types=[
        pltpu.VMEM((GD, KD), jnp.int32),
        pltpu.VMEM((KD, HC), jnp.float32),
        pltpu.VMEM_SHARED((N, HC), jnp.float32),
    ],
)
def _deg_pass(dstr, out, di, ones_b, acc):
    cid = lax.axis_index("c")
    sid = lax.axis_index("s")
    row0 = sid * RPT

    one16 = jnp.ones((16,), jnp.float32)
    zero16 = jnp.zeros((16,), jnp.float32)

    def zrow(i, carry):
        r = i // (HC // 16)
        c = (i % (HC // 16)) * 16
        ones_b[r, pl.ds(c, 16)] = zero16
        return carry

    lax.fori_loop(0, KD * (HC // 16), zrow, 0)
    for j in range(RPT // KD):
        pltpu.sync_copy(ones_b, acc.at[pl.ds(row0 + j * KD, KD)])
    rem = RPT - (RPT // KD) * K
    if rem:
        pltpu.sync_copy(ones_b.at[pl.ds(0, rem)],
                        acc.at[pl.ds(row0 + (RPT // KD) * KD, rem)])

    def orow(i, carry):
        r = i // (HC // 16)
        c = (i % (HC // 16)) * 16
        ones_b[r, pl.ds(c, 16)] = one16
        return carry

    lax.fori_loop(0, KD * (HC // 16), orow, 0)
    plsc.subcore_barrier()

    def chunk(c, carry):
        pltpu.sync_copy(ones_b, acc.at[di.at[c]], add=True)
        return carry

    for g in range(NGD):
        pltpu.sync_copy(dstr.at[sid, g], di)
        lax.fori_loop(0, GD, chunk, 0)
    plsc.subcore_barrier()
    pltpu.sync_copy(acc.at[pl.ds(row0, RPT)], out.at[cid, pl.ds(row0, RPT)])


def _whole(shape):
    nd = len(shape)
    return pl.BlockSpec(shape, lambda i, _n=nd: (0,) * _n)


def _blocked():
    return pl.BlockSpec((BLK, H), lambda i: (i, 0))


def _split_in():
    return pl.BlockSpec((NCORES, BLK, HC), lambda i: (0, i, 0))


def _hc_blocked():
    return pl.BlockSpec((BLK, HC), lambda i: (i, 0))


def _dotbf(a, b):
    return jnp.dot(a.astype(jnp.bfloat16), b.astype(jnp.bfloat16),
                   preferred_element_type=jnp.float32)


def _dot_exact(a, b):
    a = a.astype(jnp.float32)
    hi = a.astype(jnp.bfloat16)
    mid = (a - hi.astype(jnp.float32)).astype(jnp.bfloat16)
    lo = (a - hi.astype(jnp.float32)
          - mid.astype(jnp.float32)).astype(jnp.bfloat16)
    bb = b.astype(jnp.bfloat16)
    f32 = jnp.float32
    return (jnp.dot(hi, bb, preferred_element_type=f32)
            + jnp.dot(mid, bb, preferred_element_type=f32)
            + jnp.dot(lo, bb, preferred_element_type=f32))


def _matin_body(x_ref, win, bin_, t_o):
    t_o[...] = _dotbf(x_ref[...], win[...]) + bin_[...]


_matin = pl.pallas_call(
    _matin_body,
    grid=(NBLK,),
    in_specs=[
        pl.BlockSpec((BLK, 10), lambda i: (i, 0)),
        _whole((10, H)),
        _whole((H,)),
    ],
    out_specs=[_blocked()],
    out_shape=[jax.ShapeDtypeStruct((N, H), jnp.float32)],
    compiler_params=pltpu.CompilerParams(dimension_semantics=("parallel",)),
)


def _tables_body(h_ref, a0, a1, b0, b1_, bm1a, bm1b, wo, bo_ref,
                 ha0_o, ha1_o, hb0_o, hb1_o, lg_o):
    h = h_ref[...]
    ha0_o[...] = _dotbf(h, a0[...])
    ha1_o[...] = _dotbf(h, a1[...])
    hb0_o[...] = _dotbf(h, b0[...]) + bm1a[...]
    hb1_o[...] = _dotbf(h, b1_[...]) + bm1b[...]
    lg_o[...] = _dotbf(h, wo[...]) + bo_ref[...]


_tables = pl.pallas_call(
    _tables_body,
    grid=(NBLK,),
    in_specs=[
        _blocked(),
        _whole((H, HC)), _whole((H, HC)),
        _whole((H, HC)), _whole((H, HC)),
        _whole((HC,)), _whole((HC,)),
        _whole((H, 9)), _whole((9,)),
    ],
    out_specs=[_hc_blocked(), _hc_blocked(), _hc_blocked(), _hc_blocked(),
               pl.BlockSpec((BLK, 9), lambda i: (i, 0))],
    out_shape=[
        jax.ShapeDtypeStruct((N, HC), jnp.float32),
        jax.ShapeDtypeStruct((N, HC), jnp.float32),
        jax.ShapeDtypeStruct((N, HC), jnp.float32),
        jax.ShapeDtypeStruct((N, HC), jnp.float32),
        jax.ShapeDtypeStruct((N, 9), jnp.float32),
    ],
    compiler_params=pltpu.CompilerParams(dimension_semantics=("parallel",)),
)


def _gmm_body(h_ref, r2, deg,
              wm2a, wm2b, bm2_ref,
              wir, wiz, win_, bir, biz, bin2,
              whr, whz, whn, bhr, bhz, bhn,
              sr_o, sz_o, in_o, hn_o):
    h = h_ref[...]
    dcol = deg[...][:, :1]
    agg = _dot_exact(r2[0], wm2a[...])
    agg += _dot_exact(r2[1], wm2b[...])
    agg += dcol * bm2_ref[...]
    i_r = _dotbf(agg, wir[...]) + bir[...]
    i_z = _dotbf(agg, wiz[...]) + biz[...]
    i_n = _dotbf(agg, win_[...]) + bin2[...]
    h_r = _dotbf(h, whr[...]) + bhr[...]
    h_z = _dotbf(h, whz[...]) + bhz[...]
    h_n = _dotbf(h, whn[...]) + bhn[...]
    sr_o[...] = i_r + h_r
    sz_o[...] = i_z + h_z
    in_o[...] = i_n
    hn_o[...] = h_n


_gmm = pl.pallas_call(
    _gmm_body,
    grid=(NBLK,),
    in_specs=[
        _blocked(),
        _split_in(),
        pl.BlockSpec((BLK, 16), lambda i: (i, 0)),
        _whole((HC, H)), _whole((HC, H)), _whole((H,)),
        _whole((H, H)), _whole((H, H)), _whole((H, H)),
        _whole((H,)), _whole((H,)), _whole((H,)),
        _whole((H, H)), _whole((H, H)), _whole((H, H)),
        _whole((H,)), _whole((H,)), _whole((H,)),
    ],
    out_specs=[_blocked(), _blocked(), _blocked(), _blocked()],
    out_shape=[
        jax.ShapeDtypeStruct((N, H), jnp.float32),
        jax.ShapeDtypeStruct((N, H), jnp.float32),
        jax.ShapeDtypeStruct((N, H), jnp.float32),
        jax.ShapeDtypeStruct((N, H), jnp.float32),
    ],
    compiler_params=pltpu.CompilerParams(dimension_semantics=("parallel",)),
)


def _gruf_body(h_ref, r2, deg,
               wm2a, wm2b, bm2_ref,
               wir, wiz, win_, bir, biz, bin2,
               whr, whz, whn, bhr, bhz, bhn,
               g2, b2, a0, a1, b0, b1_, bm1a, bm1b, wo, bo_ref,
               h_o, ha0_o, ha1_o, hb0_o, hb1_o, lg_o):
    h = h_ref[...]
    dcol = deg[...][:, :1]
    agg = _dot_exact(r2[0], wm2a[...])
    agg += _dot_exact(r2[1], wm2b[...])
    agg += dcol * bm2_ref[...]
    s_r = _dotbf(agg, wir[...]) + bir[...] + _dotbf(h, whr[...]) + bhr[...]
    s_z = _dotbf(agg, wiz[...]) + biz[...] + _dotbf(h, whz[...]) + bhz[...]
    i_n = _dotbf(agg, win_[...]) + bin2[...]
    h_n = _dotbf(h, whn[...]) + bhn[...]
    r = jax.nn.sigmoid(s_r)
    z = jax.nn.sigmoid(s_z)
    n = jnp.tanh(i_n + r * h_n)
    hnew = (1.0 - z) * n + z * h
    m = jnp.mean(hnew, axis=-1, keepdims=True)
    v = jnp.mean((hnew - m) ** 2, axis=-1, keepdims=True)
    hn = (hnew - m) / jnp.sqrt(v + LN_EPS) * g2[...] + b2[...]
    h_o[...] = hn
    ha0_o[...] = _dotbf(hn, a0[...])
    ha1_o[...] = _dotbf(hn, a1[...])
    hb0_o[...] = _dotbf(hn, b0[...]) + bm1a[...]
    hb1_o[...] = _dotbf(hn, b1_[...]) + bm1b[...]
    lg_o[...] = _dotbf(hn, wo[...]) + bo_ref[...]


_gruf = pl.pallas_call(
    _gruf_body,
    grid=(NBLK,),
    in_specs=[
        _blocked(),
        _split_in(),
        pl.BlockSpec((BLK, 16), lambda i: (i, 0)),
        _whole((HC, H)), _whole((HC, H)), _whole((H,)),
        _whole((H, H)), _whole((H, H)), _whole((H, H)),
        _whole((H,)), _whole((H,)), _whole((H,)),
        _whole((H, H)), _whole((H, H)), _whole((H, H)),
        _whole((H,)), _whole((H,)), _whole((H,)),
        _whole((H,)), _whole((H,)),
        _whole((H, HC)), _whole((H, HC)),
        _whole((H, HC)), _whole((H, HC)),
        _whole((HC,)), _whole((HC,)),
        _whole((H, 9)), _whole((9,)),
    ],
    out_specs=[_blocked(), _hc_blocked(), _hc_blocked(), _hc_blocked(),
               _hc_blocked(), pl.BlockSpec((BLK, 9), lambda i: (i, 0))],
    out_shape=[
        jax.ShapeDtypeStruct((N, H), jnp.float32),
        jax.ShapeDtypeStruct((N, HC), jnp.float32),
        jax.ShapeDtypeStruct((N, HC), jnp.float32),
        jax.ShapeDtypeStruct((N, HC), jnp.float32),
        jax.ShapeDtypeStruct((N, HC), jnp.float32),
        jax.ShapeDtypeStruct((N, 9), jnp.float32),
    ],
    compiler_params=pltpu.CompilerParams(dimension_semantics=("parallel",)),
)


def _lnorm(t, g, b):
    m = jnp.mean(t, axis=-1, keepdims=True)
    v = jnp.var(t, axis=-1, keepdims=True)
    return (t - m) / jnp.sqrt(v + LN_EPS) * g + b


def _pad_cols(w, total=2 * HC):
    return jnp.pad(w, ((0, 0), (0, total - w.shape[1])))


def kernel(x, edge_index, W_in, b_in, ln1_g, ln1_b, pos, Wm1, bm1, Wm2, bm2,
           Wih, bih, Whh, bhh, ln2_g, ln2_b, Wo, bo):
    n_graphs = N // 81
    srcr = edge_index[1].reshape(NSUB, NGRP, G, K)
    dstr = edge_index[0].reshape(NSUB, NGRP, G, K)
    dstr_d = edge_index[0].reshape(NSUB, NGD, GD, KD)
    a_p = _pad_cols(Wm1[:H])
    b_p = _pad_cols(Wm1[H:])
    a0, a1 = a_p[:, :HC], a_p[:, HC:]
    b0, b1_ = b_p[:, :HC], b_p[:, HC:]
    bm1_p = jnp.pad(bm1, (0, 2 * HC - H))
    bm1a, bm1b = bm1_p[:HC], bm1_p[HC:]
    wm2_r = Wm2.astype(jnp.bfloat16).astype(jnp.float32)
    wm2a = wm2_r[:HC]
    wm2b = jnp.pad(wm2_r[HC:], ((0, 2 * HC - H), (0, 0)))
    wir, wiz, win_ = Wih[:, :H], Wih[:, H:2 * H], Wih[:, 2 * H:]
    bir, biz, bin2 = bih[:H], bih[H:2 * H], bih[2 * H:]
    whr, whz, whn = Whh[:, :H], Whh[:, H:2 * H], Whh[:, 2 * H:]
    bhr, bhz, bhn = bhh[:H], bhh[H:2 * H], bhh[2 * H:]

    deg = _deg_pass(dstr_d)[0, :, :16]

    (t0,) = _matin(x, W_in, b_in)
    h = _lnorm(t0, ln1_g, ln1_b) + jnp.tile(pos, (n_graphs, 1))
    ha0, ha1, hb0, hb1, _ = _tables(h, a0, a1, b0, b1_, bm1a, bm1b, Wo, bo)
    logits = []
    for _ in range(NSTEPS):
        r2 = _edge_pass(ha0, ha1, hb0, hb1, srcr, dstr)
        h, ha0, ha1, hb0, hb1, lg = _gruf(
            h, r2, deg, wm2a, wm2b, bm2,
            wir, wiz, win_, bir, biz, bin2,
            whr, whz, whn, bhr, bhz, bhn,
            ln2_g, ln2_b, a0, a1, b0, b1_, bm1a, bm1b, Wo, bo)
        logits.append(lg)
    return logits[-1], jnp.stack(logits)
